# baseline (speedup 1.0000x reference)
import jax
import jax.numpy as jnp
from jax import lax
from jax.experimental import pallas as pl
from jax.experimental.pallas import tpu as pltpu
from jax.experimental.pallas import tpu_sc as plsc

_B = 16384
_V = 1000000
_D = 16
_NS = 26
_ND = 13
_F = _NS + _ND

_NC = 2
_NSUB = 16
_NW = _NC * _NSUB
_BPW = _B // _NW
_CB = 128
_NCHUNK = _BPW // _CB
_IPC = _CB * _F

def _y1_body(spt_hbm, det_hbm, emb1_hbm, w1_hbm, y1_hbm,
             idx_v, e1_v, df_v, y1_v, w1_v, semi, sem1):
    wid = lax.axis_index("s") * _NC + lax.axis_index("c")

    pltpu.sync_copy(w1_hbm, w1_v)
    w1_vec = w1_v[...]

    def chunk(c, carry):
        b0 = wid * _BPW + c * _CB
        idescs = []
        for j in range(_NS):
            idescs.append(pltpu.async_copy(
                spt_hbm.at[pl.ds(j * _B + b0, _CB)],
                idx_v.at[pl.ds(j * _CB, _CB)], semi))
        for jd in range(_ND):
            idescs.append(pltpu.async_copy(
                det_hbm.at[pl.ds(jd * _B + b0, _CB)],
                idx_v.at[pl.ds((_NS + jd) * _CB, _CB)], semi))
        for d in idescs:
            d.wait()

        descs = []
        for j in range(_F):
            sl = pl.ds(j * _CB, _CB)
            descs.append(pltpu.async_copy(
                emb1_hbm.at[idx_v.at[sl]], e1_v.at[sl], sem1))

        def conv_grp(g, carry2):
            for jd in range(_ND):
                sl_i = pl.ds((_NS + jd) * _CB + g * 16, 16)
                sl_o = pl.ds(jd * _CB + g * 16, 16)
                df_v[sl_o] = idx_v[sl_i].astype(jnp.float32)
            return carry2

        lax.fori_loop(0, _CB // 16, conv_grp, 0)

        for d in descs:
            d.wait()

        def fo_grp(g, carry2):
            acc1 = e1_v[pl.ds(g * 16, 16)]
            for j in range(1, _F):
                acc1 = acc1 + e1_v[pl.ds(j * _CB + g * 16, 16)]
            for jd in range(_ND):
                acc1 = acc1 + df_v[pl.ds(jd * _CB + g * 16, 16)] * w1_vec[jd]
            y1_v[pl.ds(g * 16, 16)] = acc1
            return carry2

        lax.fori_loop(0, _CB // 16, fo_grp, 0)
        pltpu.sync_copy(y1_v, y1_hbm.at[pl.ds(b0, _CB)])
        return carry

    lax.fori_loop(0, _NCHUNK, chunk, 0)


def _sc_body(spt_hbm, det_hbm, emb_hbm, w_hbm, y2_hbm,
             idx_v, rows_v, dfb_v, t_v, y2_v,
             w_v, semi, sem):
    wid = lax.axis_index("s") * _NC + lax.axis_index("c")
    lanes = lax.iota(jnp.int32, 16)
    lanes16 = lanes * _D

    pltpu.sync_copy(w_hbm, w_v)
    w_rows = [w_v[pl.ds(j * _D, _D)] for j in range(_ND)]
    w2_rows = [w * w for w in w_rows]

    def chunk(c, carry):
        b0 = wid * _BPW + c * _CB

        idescs = []
        for j in range(_NS):
            idescs.append(pltpu.async_copy(
                spt_hbm.at[pl.ds(j * _B + b0, _CB)],
                idx_v.at[pl.ds(j * _CB, _CB)], semi))
        for jd in range(_ND):
            idescs.append(pltpu.async_copy(
                det_hbm.at[pl.ds(jd * _B + b0, _CB)],
                idx_v.at[pl.ds((_NS + jd) * _CB, _CB)], semi))
        for d in idescs:
            d.wait()

        descs = []
        for j in range(_F):
            sl = pl.ds(j * _CB, _CB)
            descs.append(pltpu.async_copy(
                emb_hbm.at[idx_v.at[sl]], rows_v.at[sl], sem))

        def conv_grp(g, carry2):
            for jd in range(_ND):
                sl_i = pl.ds((_NS + jd) * _CB + g * 16, 16)
                cvec = idx_v[sl_i].astype(jnp.float32)
                plsc.store_scatter(dfb_v, [lanes16 + (g * 256 + jd)], cvec)
            return carry2

        lax.fori_loop(0, _CB // 16, conv_grp, 0)

        for d in descs:
            d.wait()

        def so_row(b, carry2):
            v = rows_v[b]
            acc = v
            acc2 = v * v
            for j in range(1, _F):
                v = rows_v[j * _CB + b]
                acc = acc + v
                acc2 = acc2 + v * v
            dfv = dfb_v[pl.ds(b * _D, _D)]
            for jd in range(_ND):
                dfs = dfv[jd]
                acc = acc + dfs * w_rows[jd]
                acc2 = acc2 + (dfs * dfs) * w2_rows[jd]
            t_v[pl.ds(b * _D, _D)] = acc * acc - acc2
            return carry2

        lax.fori_loop(0, _CB, so_row, 0, unroll=4)

        def fo_grp(g, carry2):
            tl = lanes16 + g * (16 * _D)
            acc2 = plsc.load_gather(t_v, [tl])
            for d in range(1, _D):
                acc2 = acc2 + plsc.load_gather(t_v, [tl + d])
            y2_v[pl.ds(g * 16, 16)] = 0.5 * acc2
            return carry2

        lax.fori_loop(0, _CB // 16, fo_grp, 0)

        pltpu.sync_copy(y2_v, y2_hbm.at[pl.ds(b0, _CB)])
        return carry

    lax.fori_loop(0, _NCHUNK, chunk, 0)


@jax.jit
def kernel(sparse_inputs, dense_inputs, embedding_one, embedding,
           dense_w_one, dense_w):
    spt = jnp.transpose(sparse_inputs.astype(jnp.int32)).reshape(-1)
    det = jnp.transpose(dense_inputs.astype(jnp.int32)).reshape(-1)
    e1f = jnp.transpose(embedding_one).reshape(-1)
    w1p = jnp.pad(dense_w_one.astype(jnp.float32), (0, 3))
    wf = dense_w.astype(jnp.float32).reshape(_ND * _D)

    mesh = plsc.VectorSubcoreMesh(
        core_axis_name="c", subcore_axis_name="s",
        num_cores=_NC, num_subcores=_NSUB)

    y1_fn = pl.kernel(
        _y1_body,
        out_type=jax.ShapeDtypeStruct((_B,), jnp.float32),
        mesh=mesh,
        scratch_types=[
            pltpu.VMEM((_IPC,), jnp.int32),
            pltpu.VMEM((_IPC,), jnp.float32),
            pltpu.VMEM((_ND * _CB,), jnp.float32),
            pltpu.VMEM((_CB,), jnp.float32),
            pltpu.VMEM((16,), jnp.float32),
            pltpu.SemaphoreType.DMA,
            pltpu.SemaphoreType.DMA,
        ],
        compiler_params=pltpu.CompilerParams(
            needs_layout_passes=False, use_tc_tiling_on_sc=False),
    )

    sc_fn = pl.kernel(
        _sc_body,
        out_type=jax.ShapeDtypeStruct((_B,), jnp.float32),
        mesh=mesh,
        scratch_types=[
            pltpu.VMEM((_IPC,), jnp.int32),
            pltpu.VMEM((_IPC, _D), jnp.float32),
            pltpu.VMEM((_CB * _D,), jnp.float32),
            pltpu.VMEM((_CB * _D,), jnp.float32),
            pltpu.VMEM((_CB,), jnp.float32),
            pltpu.VMEM((_ND * _D,), jnp.float32),
            pltpu.SemaphoreType.DMA,
            pltpu.SemaphoreType.DMA,
        ],
        compiler_params=pltpu.CompilerParams(
            needs_layout_passes=False, use_tc_tiling_on_sc=False),
    )

    y1 = y1_fn(spt, det, e1f, w1p)
    y2 = sc_fn(spt, det, embedding, wf)
    return (y1.reshape(_B, 1), y2.reshape(_B, 1))


# speedup vs baseline: 1.0211x; 1.0211x over previous
import jax
import jax.numpy as jnp
from jax import lax
from jax.experimental import pallas as pl
from jax.experimental.pallas import tpu as pltpu
from jax.experimental.pallas import tpu_sc as plsc

_B = 16384
_V = 1000000
_D = 16
_NS = 26
_ND = 13
_F = _NS + _ND

_NC = 2
_NSUB = 16
_NW = _NC * _NSUB
_BPW = _B // _NW
_CB = 128
_NCHUNK = _BPW // _CB
_IPC = _CB * _F
_CB2 = 64
_NCHUNK2 = _BPW // _CB2

def _y1_body(spt_hbm, det_hbm, emb1_hbm, w1_hbm, y1_hbm,
             idx_v, e1_v, df_v, y1_v, w1_v, semi, sem1):
    wid = lax.axis_index("s") * _NC + lax.axis_index("c")

    pltpu.sync_copy(w1_hbm, w1_v)
    w1_vec = w1_v[...]

    def chunk(c, carry):
        b0 = wid * _BPW + c * _CB
        idescs = []
        for j in range(_NS):
            idescs.append(pltpu.async_copy(
                spt_hbm.at[pl.ds(j * _B + b0, _CB)],
                idx_v.at[pl.ds(j * _CB, _CB)], semi))
        for jd in range(_ND):
            idescs.append(pltpu.async_copy(
                det_hbm.at[pl.ds(jd * _B + b0, _CB)],
                idx_v.at[pl.ds((_NS + jd) * _CB, _CB)], semi))
        for d in idescs:
            d.wait()

        descs = []
        for j in range(_F):
            sl = pl.ds(j * _CB, _CB)
            descs.append(pltpu.async_copy(
                emb1_hbm.at[idx_v.at[sl]], e1_v.at[sl], sem1))

        def conv_grp(g, carry2):
            for jd in range(_ND):
                sl_i = pl.ds((_NS + jd) * _CB + g * 16, 16)
                sl_o = pl.ds(jd * _CB + g * 16, 16)
                df_v[sl_o] = idx_v[sl_i].astype(jnp.float32)
            return carry2

        lax.fori_loop(0, _CB // 16, conv_grp, 0)

        for d in descs:
            d.wait()

        def fo_grp(g, carry2):
            acc1 = e1_v[pl.ds(g * 16, 16)]
            for j in range(1, _F):
                acc1 = acc1 + e1_v[pl.ds(j * _CB + g * 16, 16)]
            for jd in range(_ND):
                acc1 = acc1 + df_v[pl.ds(jd * _CB + g * 16, 16)] * w1_vec[jd]
            y1_v[pl.ds(g * 16, 16)] = acc1
            return carry2

        lax.fori_loop(0, _CB // 16, fo_grp, 0)
        pltpu.sync_copy(y1_v, y1_hbm.at[pl.ds(b0, _CB)])
        return carry

    lax.fori_loop(0, _NCHUNK, chunk, 0)


def _sc_body(spt_hbm, det_hbm, emb_hbm, w_hbm, y2_hbm,
             idx_v, rows_v, dfb_v, t_v, y2_v,
             w_v, semi, sem0, sem1):
    wid = lax.axis_index("s") * _NC + lax.axis_index("c")
    lanes = lax.iota(jnp.int32, 16)
    lanes16 = lanes * _D
    sems = [sem0, sem1]

    pltpu.sync_copy(w_hbm, w_v)
    w_rows = [w_v[pl.ds(j * _D, _D)] for j in range(_ND)]
    w2_rows = [w * w for w in w_rows]

    b0w = wid * _BPW
    idescs = []
    for j in range(_NS):
        idescs.append(pltpu.async_copy(
            spt_hbm.at[pl.ds(j * _B + b0w, _BPW)],
            idx_v.at[pl.ds(j * _BPW, _BPW)], semi))
    for jd in range(_ND):
        idescs.append(pltpu.async_copy(
            det_hbm.at[pl.ds(jd * _B + b0w, _BPW)],
            idx_v.at[pl.ds((_NS + jd) * _BPW, _BPW)], semi))
    for d in idescs:
        d.wait()

    def fire(c, buf):
        descs = []
        for j in range(_F):
            descs.append(pltpu.async_copy(
                emb_hbm.at[idx_v.at[pl.ds(j * _BPW + c * _CB2, _CB2)]],
                rows_v.at[pl.ds((buf * _F + j) * _CB2, _CB2)], sems[buf]))
        return descs

    def compute(c, buf):
        b0 = wid * _BPW + c * _CB2
        rbase = buf * _F * _CB2

        def conv_grp(g, carry2):
            for jd in range(_ND):
                sl_i = pl.ds((_NS + jd) * _BPW + c * _CB2 + g * 16, 16)
                cvec = idx_v[sl_i].astype(jnp.float32)
                plsc.store_scatter(dfb_v, [lanes16 + (g * 256 + jd)], cvec)
            return carry2

        lax.fori_loop(0, _CB2 // 16, conv_grp, 0)

        def so_row(b, carry2):
            v = rows_v[rbase + b]
            acc = v
            acc2 = v * v
            for j in range(1, _F):
                v = rows_v[rbase + j * _CB2 + b]
                acc = acc + v
                acc2 = acc2 + v * v
            dfv = dfb_v[pl.ds(b * _D, _D)]
            for jd in range(_ND):
                dfs = dfv[jd]
                acc = acc + dfs * w_rows[jd]
                acc2 = acc2 + (dfs * dfs) * w2_rows[jd]
            t_v[pl.ds(b * _D, _D)] = acc * acc - acc2
            return carry2

        lax.fori_loop(0, _CB2, so_row, 0, unroll=4)

        def fo_grp(g, carry2):
            tl = lanes16 + g * (16 * _D)
            acc2 = plsc.load_gather(t_v, [tl])
            for d in range(1, _D):
                acc2 = acc2 + plsc.load_gather(t_v, [tl + d])
            y2_v[pl.ds(g * 16, 16)] = 0.5 * acc2
            return carry2

        lax.fori_loop(0, _CB2 // 16, fo_grp, 0)
        pltpu.sync_copy(y2_v.at[pl.ds(0, _CB2)], y2_hbm.at[pl.ds(b0, _CB2)])

    d0 = fire(0, 0)
    for c in range(_NCHUNK2):
        buf = c % 2
        for d in (d0 if c % 2 == 0 else d1):
            d.wait()
        if c + 1 < _NCHUNK2:
            if buf == 0:
                d1 = fire(c + 1, 1)
            else:
                d0 = fire(c + 1, 0)
        compute(c, buf)


@jax.jit
def kernel(sparse_inputs, dense_inputs, embedding_one, embedding,
           dense_w_one, dense_w):
    spt = jnp.transpose(sparse_inputs.astype(jnp.int32)).reshape(-1)
    det = jnp.transpose(dense_inputs.astype(jnp.int32)).reshape(-1)
    e1f = jnp.transpose(embedding_one).reshape(-1)
    w1p = jnp.pad(dense_w_one.astype(jnp.float32), (0, 3))
    wf = dense_w.astype(jnp.float32).reshape(_ND * _D)

    mesh = plsc.VectorSubcoreMesh(
        core_axis_name="c", subcore_axis_name="s",
        num_cores=_NC, num_subcores=_NSUB)

    y1_fn = pl.kernel(
        _y1_body,
        out_type=jax.ShapeDtypeStruct((_B,), jnp.float32),
        mesh=mesh,
        scratch_types=[
            pltpu.VMEM((_IPC,), jnp.int32),
            pltpu.VMEM((_IPC,), jnp.float32),
            pltpu.VMEM((_ND * _CB,), jnp.float32),
            pltpu.VMEM((_CB,), jnp.float32),
            pltpu.VMEM((16,), jnp.float32),
            pltpu.SemaphoreType.DMA,
            pltpu.SemaphoreType.DMA,
        ],
        compiler_params=pltpu.CompilerParams(
            needs_layout_passes=False, use_tc_tiling_on_sc=False),
    )

    sc_fn = pl.kernel(
        _sc_body,
        out_type=jax.ShapeDtypeStruct((_B,), jnp.float32),
        mesh=mesh,
        scratch_types=[
            pltpu.VMEM((_F * _BPW,), jnp.int32),
            pltpu.VMEM((2 * _F * _CB2, _D), jnp.float32),
            pltpu.VMEM((_CB2 * _D,), jnp.float32),
            pltpu.VMEM((_CB2 * _D,), jnp.float32),
            pltpu.VMEM((_CB2,), jnp.float32),
            pltpu.VMEM((_ND * _D,), jnp.float32),
            pltpu.SemaphoreType.DMA,
            pltpu.SemaphoreType.DMA,
            pltpu.SemaphoreType.DMA,
        ],
        compiler_params=pltpu.CompilerParams(
            needs_layout_passes=False, use_tc_tiling_on_sc=False),
    )

    y1 = y1_fn(spt, det, e1f, w1p)
    y2 = sc_fn(spt, det, embedding, wf)
    return (y1.reshape(_B, 1), y2.reshape(_B, 1))
